# baseline (speedup 1.0000x reference)
import math

import jax
import jax.numpy as jnp
from jax.experimental import pallas as pl
from jax.experimental.pallas import tpu as pltpu

N_NODES = 19
WINDOW = 100
STRIDE = 50
PB = 8


def _win_kernel(x_hbm, out_hbm, xbufs, obufs, in_sems, out_sems):
    B = x_hbm.shape[0]
    T = x_hbm.shape[1]
    W = (T - WINDOW) // STRIDE + 1
    N = N_NODES
    R = W * N

    def copy_in(b):
        return pltpu.make_async_copy(
            x_hbm.at[b], xbufs.at[b % 3], in_sems.at[b % 3]
        )

    def copy_out(b):
        return pltpu.make_async_copy(
            obufs.at[b % 2], out_hbm.at[pl.ds(b * R, R)], out_sems.at[b % 2]
        )

    copy_in(0).start()
    copy_in(1).start()

    def body(b, _):
        copy_in(b).wait()

        @pl.when(b + 2 < B)
        def _():
            copy_in(b + 2).start()

        @pl.when(b >= 2)
        def _():
            copy_out(b - 2).wait()

        xt = xbufs[b % 3].T
        ob = obufs.at[b % 2]
        for w in range(W):
            ob[pl.ds(w * N, N), :] = xt[:, w * STRIDE : w * STRIDE + WINDOW]
        copy_out(b).start()
        return ()

    jax.lax.fori_loop(0, B, body, (), unroll=False)
    copy_out(B - 2).wait()
    copy_out(B - 1).wait()


def _edge_kernel(pre_ref, ewrep_ref, eib_ref, ewb_ref, gstep: int, n: int):
    j = pl.program_id(0)
    eib_ref[...] = pre_ref[...] + j * (gstep * n)
    ewb_ref[...] = ewrep_ref[...]


def _bv_kernel(bv_ref):
    r = jax.lax.broadcasted_iota(jnp.int32, bv_ref.shape, 0)
    bv_ref[...] = r // N_NODES


def kernel(x, edge_index, edge_weight):
    B, T, N = x.shape
    W = (T - WINDOW) // STRIDE + 1
    G = B * W
    E = edge_index.shape[1]

    x_batched = pl.pallas_call(
        _win_kernel,
        in_specs=[pl.BlockSpec(memory_space=pl.ANY)],
        out_specs=pl.BlockSpec(memory_space=pl.ANY),
        out_shape=jax.ShapeDtypeStruct((G * N, WINDOW), jnp.float32),
        scratch_shapes=[
            pltpu.VMEM((3, T, N), jnp.float32),
            pltpu.VMEM((2, W * N, WINDOW), jnp.float32),
            pltpu.SemaphoreType.DMA((3,)),
            pltpu.SemaphoreType.DMA((2,)),
        ],
    )(x)

    ei = edge_index.astype(jnp.int32)
    CE = G * E
    lcm = (E * 1024) // math.gcd(E, 1024)
    gstep = lcm // E
    CB = lcm
    nblk = -(-CE // CB)
    col = jnp.arange(CB, dtype=jnp.int32)
    pre = jnp.tile(ei, (1, gstep)) + (col // E * N)[None, :]
    ewrep = jnp.tile(edge_weight, gstep)

    ei_b, ew_b = pl.pallas_call(
        lambda p, w, o1, o2: _edge_kernel(p, w, o1, o2, gstep, N),
        grid=(nblk,),
        in_specs=[
            pl.BlockSpec((2, CB), lambda j: (0, 0)),
            pl.BlockSpec((CB,), lambda j: (0,)),
        ],
        out_specs=[
            pl.BlockSpec((2, CB), lambda j: (0, j)),
            pl.BlockSpec((CB,), lambda j: (j,)),
        ],
        out_shape=[
            jax.ShapeDtypeStruct((2, CE), jnp.int32),
            jax.ShapeDtypeStruct((CE,), jnp.float32),
        ],
    )(pre, ewrep)

    batch_vec = pl.pallas_call(
        _bv_kernel,
        out_shape=jax.ShapeDtypeStruct((G * N,), jnp.int32),
    )()

    return x_batched, ei_b, ew_b, batch_vec
